# Initial kernel scaffold; baseline (speedup 1.0000x reference)
import jax
import jax.numpy as jnp
from jax.experimental import pallas as pl


def kernel(objectness, box_regression, anchors):
    raise NotImplementedError("write your pallas kernel here")


# speedup vs baseline: 88.4230x; 88.4230x over previous
import functools
import math

import jax
import jax.numpy as jnp
from jax import lax
from jax.experimental import pallas as pl
from jax.experimental.pallas import tpu as pltpu
from jax.experimental.pallas import tpu_sc as plsc

IMW, IMH = 1066, 800
PRE_N = 2000
PRE = 2048
POST = 1000
POST_PAD = 1024
NMS_T = 0.7
XCLIP = math.log(1000.0 / 16.0)
MPAD = 32768
SROWS = MPAD // 128
TROWS = PRE // 128
NEG = -1e9


def _sort_body(s_ref, os_ref, oi_ref):
    n = pl.program_id(0)
    s = s_ref[0]
    pos = lax.broadcasted_iota(jnp.int32, (SROWS, 128), 0) * 128 + \
        lax.broadcasted_iota(jnp.int32, (SROWS, 128), 1)
    idx = pos
    for ke in range(1, 16):
        kbit = 1 << ke
        desc = (pos & kbit) == 0
        for je in reversed(range(ke)):
            j = 1 << je
            bit0 = (pos & j) == 0
            if j < 128:
                ps = jnp.where(bit0, jnp.roll(s, -j, axis=1), jnp.roll(s, j, axis=1))
                pi = jnp.where(bit0, jnp.roll(idx, -j, axis=1), jnp.roll(idx, j, axis=1))
            else:
                dj = j // 128
                ps = jnp.where(bit0, jnp.roll(s, -dj, axis=0), jnp.roll(s, dj, axis=0))
                pi = jnp.where(bit0, jnp.roll(idx, -dj, axis=0), jnp.roll(idx, dj, axis=0))
            before = (ps > s) | ((ps == s) & (pi < idx))
            take = before == (desc == bit0)
            s = jnp.where(take, ps, s)
            idx = jnp.where(take, pi, idx)
    os_ref[0] = s[:TROWS, :]
    oi_ref[0] = idx[:TROWS, :] + n * MPAD


def _topk_sort(scores_pad):
    n = scores_pad.shape[0]
    return pl.pallas_call(
        _sort_body,
        grid=(n,),
        in_specs=[pl.BlockSpec((1, SROWS, 128), lambda i: (i, 0, 0))],
        out_specs=[
            pl.BlockSpec((1, TROWS, 128), lambda i: (i, 0, 0)),
            pl.BlockSpec((1, TROWS, 128), lambda i: (i, 0, 0)),
        ],
        out_shape=[
            jax.ShapeDtypeStruct((n, TROWS, 128), jnp.float32),
            jax.ShapeDtypeStruct((n, TROWS, 128), jnp.int32),
        ],
    )(scores_pad)


def _sc_gather(table, flat_idx):
    b_tot = flat_idx.shape[0]
    nw = 32
    bpw = b_tot // nw
    mesh = plsc.VectorSubcoreMesh(core_axis_name="c", subcore_axis_name="s")

    @functools.partial(
        pl.kernel,
        mesh=mesh,
        compiler_params=pltpu.CompilerParams(use_tc_tiling_on_sc=False),
        out_type=jax.ShapeDtypeStruct((b_tot, 16), jnp.float32),
        scratch_types=[
            pltpu.VMEM((bpw,), jnp.int32),
            pltpu.VMEM((bpw, 16), jnp.float32),
            pltpu.SemaphoreType.DMA,
        ],
    )
    def body(table_hbm, idx_hbm, out_hbm, idx_v, rows_v, sem):
        wid = lax.axis_index("s") * 2 + lax.axis_index("c")
        base = wid * bpw
        pltpu.sync_copy(idx_hbm.at[pl.ds(base, bpw)], idx_v)
        pltpu.async_copy(table_hbm.at[idx_v], rows_v, sem).wait()
        pltpu.sync_copy(rows_v, out_hbm.at[pl.ds(base, bpw)])

    return body(table, flat_idx)


def _decode_clip(dx, dy, dw, dh, ax1, ay1, ax2, ay2):
    w = ax2 - ax1 + 1.0
    h = ay2 - ay1 + 1.0
    cx = ax1 + 0.5 * w
    cy = ay1 + 0.5 * h
    pcx = dx * w + cx
    pcy = dy * h + cy
    pw = jnp.exp(jnp.minimum(dw, XCLIP)) * w
    ph = jnp.exp(jnp.minimum(dh, XCLIP)) * h
    x1 = pcx - 0.5 * pw
    y1 = pcy - 0.5 * ph
    x2 = pcx + 0.5 * pw - 1.0
    y2 = pcy + 0.5 * ph - 1.0
    x1 = jnp.clip(x1, 0.0, IMW - 1.0)
    y1 = jnp.clip(y1, 0.0, IMH - 1.0)
    x2 = jnp.clip(x2, 0.0, IMW - 1.0)
    y2 = jnp.clip(y2, 0.0, IMH - 1.0)
    return x1, y1, x2, y2


def _cumsum_lanes(x):
    lanes = lax.broadcasted_iota(jnp.int32, (1, PRE), 1)
    c = x
    sh = 1
    while sh < PRE:
        c = c + jnp.where(lanes >= sh, jnp.roll(c, sh, axis=1), 0.0)
        sh *= 2
    return c


def _post_body(gt_ref, gc_ref, sc_ref, out_ref, s_mat):
    gt = gt_ref[0]
    gc = gc_ref[0]
    scol = sc_ref[0]

    x1r, y1r, x2r, y2r = _decode_clip(
        gt[0:1, :], gt[1:2, :], gt[2:3, :], gt[3:4, :],
        gt[4:5, :], gt[5:6, :], gt[6:7, :], gt[7:8, :])
    x1c, y1c, x2c, y2c = _decode_clip(
        gc[:, 0:1], gc[:, 1:2], gc[:, 2:3], gc[:, 3:4],
        gc[:, 4:5], gc[:, 5:6], gc[:, 6:7], gc[:, 7:8])

    valid_r = ((x2r - x1r + 1.0 >= 0.0) & (y2r - y1r + 1.0 >= 0.0))
    area_r = (x2r - x1r + 1.0) * (y2r - y1r + 1.0)
    area_c = (x2c - x1c + 1.0) * (y2c - y1c + 1.0)

    blk = 256
    for b in range(PRE // blk):
        lo, hi = b * blk, (b + 1) * blk
        x1b, y1b = x1c[lo:hi, :], y1c[lo:hi, :]
        x2b, y2b = x2c[lo:hi, :], y2c[lo:hi, :]
        ix1 = jnp.maximum(x1b, x1r)
        iy1 = jnp.maximum(y1b, y1r)
        ix2 = jnp.minimum(x2b, x2r)
        iy2 = jnp.minimum(y2b, y2r)
        iw = jnp.clip(ix2 - ix1 + 1.0, 0.0, None)
        ih = jnp.clip(iy2 - iy1 + 1.0, 0.0, None)
        inter = iw * ih
        iou = inter / (area_c[lo:hi, :] + area_r - inter)
        i_io = lax.broadcasted_iota(jnp.int32, (blk, PRE), 0) + b * blk
        j_io = lax.broadcasted_iota(jnp.int32, (blk, PRE), 1)
        sup = (iou > NMS_T) & (i_io < j_io) & (i_io < PRE_N)
        s_mat[pl.ds(lo, blk), :] = jnp.where(sup, 1.0, 0.0)

    smat = s_mat[...]

    def cond(carry):
        return carry[1]

    def body(carry):
        k, _ = carry
        t = lax.dot_general(k, smat, (((1,), (0,)), ((), ())),
                            preferred_element_type=jnp.float32)
        k_new = jnp.where(t > 0.0, 0.0, 1.0)
        return k_new, jnp.any(k_new != k)

    k0 = jnp.ones((1, PRE), jnp.float32)
    k_fin, _ = lax.while_loop(cond, body, (k0, jnp.bool_(True)))

    rank_r = lax.broadcasted_iota(jnp.int32, (1, PRE), 1)
    keep = k_fin * jnp.where(valid_r & (rank_r < PRE_N), 1.0, 0.0)

    csum_k = _cumsum_lanes(keep)
    csum_n = _cumsum_lanes(1.0 - keep)
    nkept = csum_k[0, PRE - 1]
    posn = jnp.where(keep > 0.0, csum_k - 1.0, nkept + csum_n - 1.0)

    p_io = lax.broadcasted_iota(jnp.int32, (POST_PAD, PRE), 0).astype(jnp.float32)
    perm = jnp.where(p_io == posn, 1.0, 0.0)

    data = jnp.concatenate(
        [x1c, y1c, x2c, y2c, scol, jnp.zeros((PRE, 3), jnp.float32)], axis=1)
    out = lax.dot_general(perm, data, (((1,), (0,)), ((), ())),
                          precision=lax.Precision.HIGHEST,
                          preferred_element_type=jnp.float32)

    r_io = lax.broadcasted_iota(jnp.int32, (POST_PAD, 8), 0).astype(jnp.float32)
    c_io = lax.broadcasted_iota(jnp.int32, (POST_PAD, 8), 1)
    out = jnp.where((c_io == 4) & (r_io >= nkept), NEG, out)
    out_ref[0] = out


def _post(gath_t, gath, scol):
    n = gath.shape[0]
    return pl.pallas_call(
        _post_body,
        grid=(n,),
        in_specs=[
            pl.BlockSpec((1, 16, PRE), lambda i: (i, 0, 0)),
            pl.BlockSpec((1, PRE, 16), lambda i: (i, 0, 0)),
            pl.BlockSpec((1, PRE, 1), lambda i: (i, 0, 0)),
        ],
        out_specs=pl.BlockSpec((1, POST_PAD, 8), lambda i: (i, 0, 0)),
        out_shape=jax.ShapeDtypeStruct((n, POST_PAD, 8), jnp.float32),
        scratch_shapes=[pltpu.VMEM((PRE, PRE), jnp.float32)],
    )(gath_t, gath, scol)


def kernel(objectness, box_regression, anchors):
    n, a, h, w = objectness.shape
    m = a * h * w
    obj = jnp.transpose(objectness, (0, 2, 3, 1)).reshape(n, m)
    scores = jax.nn.sigmoid(obj)
    reg = jnp.transpose(box_regression.reshape(n, a, 4, h, w),
                        (0, 3, 4, 1, 2)).reshape(n, m, 4)

    s_pad = jnp.concatenate(
        [scores, jnp.full((n, MPAD - m), -1.0, jnp.float32)], axis=1)
    s_pad = s_pad.reshape(n, SROWS, 128)

    table = jnp.concatenate(
        [reg, anchors, jnp.zeros((n, m, 8), jnp.float32)], axis=2)
    table = jnp.concatenate(
        [table, jnp.zeros((n, MPAD - m, 16), jnp.float32)], axis=1)
    table = table.reshape(n * MPAD, 16)

    top_s, top_i = _topk_sort(s_pad)

    gath = _sc_gather(table, top_i.reshape(n * PRE))
    gath = gath.reshape(n, PRE, 16)
    gath_t = jnp.transpose(gath, (0, 2, 1))

    out = _post(gath_t, gath, top_s.reshape(n, PRE, 1))
    return out[:, :POST, :5]
